# baseline (speedup 1.0000x reference)
import functools

import jax
import jax.numpy as jnp
from jax import lax
from jax.experimental import pallas as pl
from jax.experimental.pallas import tpu as pltpu
from jax.experimental.pallas import tpu_sc as plsc

WIDTH = 1242
HEIGHT = 375
HW = HEIGHT * WIDTH
WPAD = 1280
HPAD = 376
MAPN = HPAD * WPAD
VALID_LIM = HEIGHT * WPAD
KPLANE = 8
CPAD = WPAD * KPLANE
ALPHA = 2.0
TLW = 4.0
DLW = 1.0
PLW = 40.0

NCORE = 2
NSUB = 16
SLICE = MAPN // NSUB


def _stage1_body(grt_ref, kmat_ref, rv_ref, tv_ref, pts_ref, gt_ref,
                 pix_ref, z_ref, psum_ref, colsq_ref):
    i = pl.program_id(0)
    r0 = rv_ref[i, 0]
    r1 = rv_ref[i, 1]
    r2 = rv_ref[i, 2]
    t0 = tv_ref[i, 0]
    t1 = tv_ref[i, 1]
    t2 = tv_ref[i, 2]
    th2 = r0 * r0 + r1 * r1 + r2 * r2
    th = jnp.sqrt(th2)
    a = jnp.sin(th) / th
    bc = (1.0 - jnp.cos(th)) / th2
    ct = 1.0 - bc * th2
    rt00 = ct + bc * r0 * r0
    rt01 = -a * r2 + bc * r0 * r1
    rt02 = a * r1 + bc * r0 * r2
    rt10 = a * r2 + bc * r1 * r0
    rt11 = ct + bc * r1 * r1
    rt12 = -a * r0 + bc * r1 * r2
    rt20 = -a * r1 + bc * r2 * r0
    rt21 = a * r0 + bc * r2 * r1
    rt22 = ct + bc * r2 * r2

    x0 = pts_ref[0, 0:1, :]
    x1 = pts_ref[0, 1:2, :]
    x2 = pts_ref[0, 2:3, :]
    x3 = pts_ref[0, 3:4, :]

    def grow(cc):
        return (grt_ref[i, cc, 0] * x0 + grt_ref[i, cc, 1] * x1
                + grt_ref[i, cc, 2] * x2 + grt_ref[i, cc, 3] * x3)

    pg0 = grow(0)
    pg1 = grow(1)
    pg2 = grow(2)
    pg3 = grow(3)

    pp0 = rt00 * x0 + rt01 * x1 + rt02 * x2 + t0 * x3
    pp1 = rt10 * x0 + rt11 * x1 + rt12 * x2 + t1 * x3
    pp2 = rt20 * x0 + rt21 * x1 + rt22 * x2 + t2 * x3
    d0 = pp0 - pg0
    d1 = pp1 - pg1
    d2 = pp2 - pg2
    d3 = x3 - pg3
    err = jnp.sqrt(d0 * d0 + d1 * d1 + d2 * d2 + d3 * d3)
    psum_ref[...] = jnp.sum(err).reshape(1, 1, 1)

    u = kmat_ref[0, 0] * pp0 + kmat_ref[0, 1] * pp1 + kmat_ref[0, 2] * pp2
    v = kmat_ref[1, 0] * pp0 + kmat_ref[1, 1] * pp1 + kmat_ref[1, 2] * pp2
    z = kmat_ref[2, 0] * pp0 + kmat_ref[2, 1] * pp1 + kmat_ref[2, 2] * pp2
    x = jnp.clip(u / z, 0.0, WIDTH - 1.0)
    y = jnp.clip(v / z, 0.0, HEIGHT - 1.0)
    xi = x.astype(jnp.int32)
    yi = jnp.where(z > 0.0, y.astype(jnp.int32), HEIGHT)
    n = pts_ref.shape[2]
    pix_ref[...] = (yi * WPAD + xi).reshape(1, 1, n)
    z_ref[...] = z.reshape(1, 1, n)

    g = gt_ref[0, 0]
    colsq_ref[...] = jnp.sum(g * g, axis=0).reshape(1, 1, WIDTH)


def _stage3_body(colsq_ref, parts_ref, psum_ref, gtt_ref, prt_ref,
                 gtr_ref, prr_ref, total_ref, tl_ref, dm_ref, pc_ref):
    b = colsq_ref.shape[0]
    tot = colsq_ref[:, 0, :]
    for q in range(KPLANE):
        tot = tot + parts_ref[:, 0, q * WPAD:q * WPAD + WIDTH]
    d = jnp.sqrt(jnp.maximum(tot, 0.0))
    dml_b = jnp.sum(d) / (WIDTH * 1.0) / b
    pcl_b = jnp.sum(psum_ref[...]) / b
    dt = prt_ref[...] - gtt_ref[...]
    dr = prr_ref[...] - gtr_ref[...]
    lt = jnp.sum(dt * dt) / b
    lr = jnp.sum(dr * dr) / b
    tl = lt + ALPHA * lr
    total_ref[...] = (TLW * tl + DLW * dml_b + PLW * pcl_b).reshape(1, 1)
    tl_ref[...] = tl.reshape(1, 1)
    dm_ref[...] = dml_b.reshape(1, 1)
    pc_ref[...] = pcl_b.reshape(1, 1)


def _make_sc_kernel(batch, n):
    chunk = n // NSUB
    rows = chunk // 128
    b_per_core = batch // NCORE
    mesh = plsc.VectorSubcoreMesh(core_axis_name="c", subcore_axis_name="s")

    @functools.partial(
        pl.kernel,
        mesh=mesh,
        out_type=(
            jax.ShapeDtypeStruct((batch, 1, CPAD), jnp.float32),
            jax.ShapeDtypeStruct((NCORE * MAPN,), jnp.float32),
            jax.ShapeDtypeStruct((NCORE * MAPN,), jnp.float32),
        ),
        scratch_types=[
            pltpu.VMEM((rows, 128), jnp.int32),
            pltpu.VMEM((rows, 128), jnp.int32),
            pltpu.VMEM((rows, 128), jnp.int32),
            pltpu.VMEM((rows, 128), jnp.int32),
            pltpu.VMEM((rows, 128), jnp.float32),
            pltpu.VMEM((rows, 128), jnp.float32),
            pltpu.VMEM((rows, 128), jnp.float32),
            pltpu.VMEM((rows, 128), jnp.float32),
            pltpu.VMEM((rows, 128), jnp.float32),
            pltpu.VMEM((rows, 128), jnp.float32),
            pltpu.VMEM((rows, 128), jnp.float32),
            pltpu.VMEM((rows, 128), jnp.float32),
            pltpu.VMEM((rows, 128), jnp.float32),
            pltpu.VMEM((CPAD,), jnp.float32),
            pltpu.VMEM((SLICE,), jnp.float32),
            pltpu.VMEM_SHARED((MAPN,), jnp.float32),
            pltpu.VMEM_SHARED((MAPN,), jnp.float32),
            pltpu.VMEM_SHARED((CPAD,), jnp.float32),
            pltpu.SemaphoreType.DMA,
        ],
    )
    def sc_kernel(pix_hbm, z_hbm, gt_hbm, out_hbm, cnt_hbm, sz_hbm,
                  pix_v, cpix_v, gidx_v, col_v, z_v, pc_v, pz_v, nc_v, nz_v,
                  cg_v, sg_v, g_v, sh_v, czero_v, zeros_v,
                  cntmap, szmap, colspm, sem):
        c = lax.axis_index("c")
        s = lax.axis_index("s")
        zero16 = jnp.zeros((16,), jnp.float32)

        def zero_body(w, carry):
            zeros_v[pl.ds(pl.multiple_of(w * 16, 16), 16)] = zero16
            return carry

        lax.fori_loop(0, SLICE // 16, zero_body, 0)

        def czero_body(w, carry):
            czero_v[pl.ds(pl.multiple_of(w * 16, 16), 16)] = zero16
            return carry

        lax.fori_loop(0, CPAD // 16, czero_body, 0)

        sl_me = pl.ds(pl.multiple_of(s * SLICE, SLICE), SLICE)
        pltpu.sync_copy(zeros_v, cntmap.at[sl_me])
        pltpu.sync_copy(zeros_v, szmap.at[sl_me])

        def batch_body(k, carry):
            b = c * b_per_core + k
            pltpu.sync_copy(pix_hbm.at[b, s], pix_v)
            pltpu.sync_copy(z_hbm.at[b, s], z_v)
            goff = b * HW
            coff = c * MAPN

            def val_body(j, carry2):
                r = j >> 3
                sl = pl.ds(pl.multiple_of((j & 7) * 16, 16), 16)
                p16 = pix_v[r, sl]
                valid = p16 < VALID_LIM
                cnt16 = jnp.where(valid, 1.0, 0.0)
                zz16 = jnp.where(valid, z_v[r, sl], 0.0)
                pc_v[r, sl] = cnt16
                pz_v[r, sl] = zz16
                nc_v[r, sl] = -cnt16
                nz_v[r, sl] = -zz16
                cpix_v[r, sl] = coff + p16
                y16 = lax.div(p16, WPAD)
                x16 = p16 - y16 * WPAD
                gidx_v[r, sl] = goff + jnp.minimum(
                    y16 * WIDTH + x16, HW - 1)
                col_v[r, sl] = x16 + WPAD * (y16 & (KPLANE - 1))
                return carry2

            lax.fori_loop(0, rows * 8, val_body, 0)

            @pl.when(s == 0)
            def _zero_cols():
                pltpu.sync_copy(czero_v, colspm)

            hs = []
            for r in range(rows):
                idx = pix_v.at[r]
                hs.append(pltpu.async_copy(pc_v.at[r], cntmap.at[idx], sem))
                hs.append(pltpu.async_copy(pz_v.at[r], szmap.at[idx], sem))
            for h in hs:
                h.wait()
            plsc.subcore_barrier()
            stage_sl = pl.ds(
                pl.multiple_of(coff + s * SLICE, SLICE), SLICE)
            h1 = pltpu.async_copy(cntmap.at[sl_me], cnt_hbm.at[stage_sl], sem)
            h2 = pltpu.async_copy(szmap.at[sl_me], sz_hbm.at[stage_sl], sem)
            h1.wait()
            h2.wait()
            plsc.subcore_barrier()
            hs = []
            for r in range(rows):
                idx = cpix_v.at[r]
                hs.append(pltpu.async_copy(cnt_hbm.at[idx], cg_v.at[r], sem))
                hs.append(pltpu.async_copy(sz_hbm.at[idx], sg_v.at[r], sem))
                hs.append(pltpu.async_copy(gt_hbm.at[gidx_v.at[r]],
                                           g_v.at[r], sem))
            for h in hs:
                h.wait()
            hs = []
            for r in range(rows):
                idx = pix_v.at[r]
                hs.append(pltpu.async_copy(nc_v.at[r], cntmap.at[idx], sem))
                hs.append(pltpu.async_copy(nz_v.at[r], szmap.at[idx], sem))
            for h in hs:
                h.wait()

            def share_body(j, carry2):
                r = j >> 3
                sl = pl.ds(pl.multiple_of((j & 7) * 16, 16), 16)
                p16 = pix_v[r, sl]
                valid = p16 < VALID_LIM
                rcp = 1.0 / jnp.maximum(cg_v[r, sl], 1.0)
                zbar = sg_v[r, sl] * rcp
                shr = zbar * (zbar - 2.0 * g_v[r, sl]) * rcp
                sh_v[r, sl] = jnp.where(valid, shr, 0.0)
                return carry2

            lax.fori_loop(0, rows * 8, share_body, 0)
            hs = []
            for r in range(rows):
                hs.append(pltpu.async_copy(sh_v.at[r],
                                           colspm.at[col_v.at[r]], sem))
            for h in hs:
                h.wait()
            plsc.subcore_barrier()

            @pl.when(s == 0)
            def _write_out():
                pltpu.sync_copy(colspm, out_hbm.at[b, 0])

            return carry

        lax.fori_loop(0, b_per_core, batch_body, 0)

    return sc_kernel


def kernel(point_clouds, gt_translation_vector, gt_rotation_vector,
           predicted_translation_vector, predicted_rotation_vector,
           gt_rt_matrix, k_matrix, gt_depth_map):
    batch = point_clouds.shape[0]
    n = point_clouds.shape[2]
    pts = jnp.transpose(point_clouds[:, 0], (0, 2, 1))

    smem = pl.BlockSpec(memory_space=pltpu.SMEM)
    pix, z, psum, colsq = pl.pallas_call(
        _stage1_body,
        grid=(batch,),
        in_specs=[
            smem,
            smem,
            smem,
            smem,
            pl.BlockSpec((1, 4, n), lambda i: (i, 0, 0)),
            pl.BlockSpec((1, 1, HEIGHT, WIDTH),
                         lambda i: (i, 0, 0, 0)),
        ],
        out_specs=[
            pl.BlockSpec((1, 1, n), lambda i: (i, 0, 0)),
            pl.BlockSpec((1, 1, n), lambda i: (i, 0, 0)),
            pl.BlockSpec((1, 1, 1), lambda i: (i, 0, 0)),
            pl.BlockSpec((1, 1, WIDTH), lambda i: (i, 0, 0)),
        ],
        out_shape=[
            jax.ShapeDtypeStruct((batch, 1, n), jnp.int32),
            jax.ShapeDtypeStruct((batch, 1, n), jnp.float32),
            jax.ShapeDtypeStruct((batch, 1, 1), jnp.float32),
            jax.ShapeDtypeStruct((batch, 1, WIDTH), jnp.float32),
        ],
    )(gt_rt_matrix, k_matrix, predicted_rotation_vector,
      predicted_translation_vector, pts, gt_depth_map)

    pix4 = pix.reshape(batch, NSUB, n // NSUB // 128, 128)
    z4 = z.reshape(batch, NSUB, n // NSUB // 128, 128)
    gt_flat = gt_depth_map.reshape(batch * HW)
    parts, _, _ = _make_sc_kernel(batch, n)(pix4, z4, gt_flat)

    outs = pl.pallas_call(
        _stage3_body,
        out_shape=[jax.ShapeDtypeStruct((1, 1), jnp.float32)] * 4,
    )(colsq, parts, psum / n,
      gt_translation_vector, predicted_translation_vector,
      gt_rotation_vector, predicted_rotation_vector)
    total, tl, dm, pc = (o.reshape(()) for o in outs)
    return (total, tl, dm, pc)


# speedup vs baseline: 1.0929x; 1.0929x over previous
import functools

import jax
import jax.numpy as jnp
from jax import lax
from jax.experimental import pallas as pl
from jax.experimental.pallas import tpu as pltpu
from jax.experimental.pallas import tpu_sc as plsc

WIDTH = 1242
HEIGHT = 375
WPAD = 1280
HPAD = 376
MAPN = HPAD * WPAD
VALID_LIM = HEIGHT * WPAD
ALPHA = 2.0
TLW = 4.0
DLW = 1.0
PLW = 40.0

NCORE = 2
NSUB = 16
SLICE = MAPN // NSUB


def _stage1_body(grt_ref, kmat_ref, rv_ref, tv_ref, pts_ref,
                 pix_ref, z_ref, psum_ref):
    i = pl.program_id(0)
    r0 = rv_ref[i, 0]
    r1 = rv_ref[i, 1]
    r2 = rv_ref[i, 2]
    t0 = tv_ref[i, 0]
    t1 = tv_ref[i, 1]
    t2 = tv_ref[i, 2]
    th2 = r0 * r0 + r1 * r1 + r2 * r2
    th = jnp.sqrt(th2)
    a = jnp.sin(th) / th
    bc = (1.0 - jnp.cos(th)) / th2
    ct = 1.0 - bc * th2
    rt00 = ct + bc * r0 * r0
    rt01 = -a * r2 + bc * r0 * r1
    rt02 = a * r1 + bc * r0 * r2
    rt10 = a * r2 + bc * r1 * r0
    rt11 = ct + bc * r1 * r1
    rt12 = -a * r0 + bc * r1 * r2
    rt20 = -a * r1 + bc * r2 * r0
    rt21 = a * r0 + bc * r2 * r1
    rt22 = ct + bc * r2 * r2

    x0 = pts_ref[0, 0:1, :]
    x1 = pts_ref[0, 1:2, :]
    x2 = pts_ref[0, 2:3, :]
    x3 = pts_ref[0, 3:4, :]

    def grow(cc):
        return (grt_ref[i, cc, 0] * x0 + grt_ref[i, cc, 1] * x1
                + grt_ref[i, cc, 2] * x2 + grt_ref[i, cc, 3] * x3)

    pg0 = grow(0)
    pg1 = grow(1)
    pg2 = grow(2)
    pg3 = grow(3)

    pp0 = rt00 * x0 + rt01 * x1 + rt02 * x2 + t0 * x3
    pp1 = rt10 * x0 + rt11 * x1 + rt12 * x2 + t1 * x3
    pp2 = rt20 * x0 + rt21 * x1 + rt22 * x2 + t2 * x3
    d0 = pp0 - pg0
    d1 = pp1 - pg1
    d2 = pp2 - pg2
    d3 = x3 - pg3
    err = jnp.sqrt(d0 * d0 + d1 * d1 + d2 * d2 + d3 * d3)
    psum_ref[...] = jnp.sum(err).reshape(1, 1, 1)

    u = kmat_ref[0, 0] * pp0 + kmat_ref[0, 1] * pp1 + kmat_ref[0, 2] * pp2
    v = kmat_ref[1, 0] * pp0 + kmat_ref[1, 1] * pp1 + kmat_ref[1, 2] * pp2
    z = kmat_ref[2, 0] * pp0 + kmat_ref[2, 1] * pp1 + kmat_ref[2, 2] * pp2
    x = jnp.clip(u / z, 0.0, WIDTH - 1.0)
    y = jnp.clip(v / z, 0.0, HEIGHT - 1.0)
    xi = x.astype(jnp.int32)
    yi = jnp.where(z > 0.0, y.astype(jnp.int32), HEIGHT)
    n = pts_ref.shape[2]
    pix_ref[...] = (yi * WPAD + xi).reshape(1, 1, n)
    z_ref[...] = z.reshape(1, 1, n)


def _stage3_body(cnt_ref, sz_ref, gt_ref, depth_ref):
    cm = cnt_ref[0, :HEIGHT, :WIDTH]
    sm = sz_ref[0, :HEIGHT, :WIDTH]
    g = gt_ref[0, 0]
    pred = jnp.where(cm > 0.5, sm / cm, 0.0)
    d = pred - g
    cs = jnp.sum(d * d, axis=0)
    depth_ref[...] = (jnp.sum(jnp.sqrt(cs)) / WIDTH).reshape(1, 1, 1)


def _stage4_body(depth_ref, psum_ref, gtt_ref, prt_ref, gtr_ref, prr_ref,
                 total_ref, tl_ref, dm_ref, pc_ref):
    b = depth_ref.shape[0]
    dml_b = jnp.sum(depth_ref[...]) / b
    pcl_b = jnp.sum(psum_ref[...]) / b
    dt = prt_ref[...] - gtt_ref[...]
    dr = prr_ref[...] - gtr_ref[...]
    lt = jnp.sum(dt * dt) / b
    lr = jnp.sum(dr * dr) / b
    tl = lt + ALPHA * lr
    total_ref[...] = (TLW * tl + DLW * dml_b + PLW * pcl_b).reshape(1, 1)
    tl_ref[...] = tl.reshape(1, 1)
    dm_ref[...] = dml_b.reshape(1, 1)
    pc_ref[...] = pcl_b.reshape(1, 1)


def _make_sc_kernel(batch, n):
    chunk = n // NSUB
    rows = chunk // 128
    b_per_core = batch // NCORE
    mesh = plsc.VectorSubcoreMesh(core_axis_name="c", subcore_axis_name="s")

    @functools.partial(
        pl.kernel,
        mesh=mesh,
        out_type=(
            jax.ShapeDtypeStruct((batch, 1, MAPN), jnp.float32),
            jax.ShapeDtypeStruct((batch, 1, MAPN), jnp.float32),
        ),
        scratch_types=[
            pltpu.VMEM((rows, 128), jnp.int32),
            pltpu.VMEM((rows, 128), jnp.float32),
            pltpu.VMEM((rows, 128), jnp.float32),
            pltpu.VMEM((rows, 128), jnp.float32),
            pltpu.VMEM((rows, 128), jnp.float32),
            pltpu.VMEM((rows, 128), jnp.float32),
            pltpu.VMEM((SLICE,), jnp.float32),
            pltpu.VMEM_SHARED((MAPN,), jnp.float32),
            pltpu.VMEM_SHARED((MAPN,), jnp.float32),
            pltpu.SemaphoreType.DMA,
        ],
    )
    def sc_kernel(pix_hbm, z_hbm, cnt_hbm, sz_hbm,
                  pix_v, z_v, pc_v, pz_v, nc_v, nz_v, zeros_v,
                  cntmap, szmap, sem):
        c = lax.axis_index("c")
        s = lax.axis_index("s")
        zero16 = jnp.zeros((16,), jnp.float32)

        def zero_body(w, carry):
            zeros_v[pl.ds(pl.multiple_of(w * 16, 16), 16)] = zero16
            return carry

        lax.fori_loop(0, SLICE // 16, zero_body, 0)
        sl_me = pl.ds(pl.multiple_of(s * SLICE, SLICE), SLICE)
        pltpu.sync_copy(zeros_v, cntmap.at[sl_me])
        pltpu.sync_copy(zeros_v, szmap.at[sl_me])
        plsc.subcore_barrier()

        def batch_body(k, carry):
            b = c * b_per_core + k
            pltpu.sync_copy(pix_hbm.at[b, s], pix_v)
            pltpu.sync_copy(z_hbm.at[b, s], z_v)

            def val_body(j, carry2):
                r = j >> 3
                sl = pl.ds(pl.multiple_of((j & 7) * 16, 16), 16)
                valid = pix_v[r, sl] < VALID_LIM
                cnt16 = jnp.where(valid, 1.0, 0.0)
                zz16 = jnp.where(valid, z_v[r, sl], 0.0)
                pc_v[r, sl] = cnt16
                pz_v[r, sl] = zz16
                nc_v[r, sl] = -cnt16
                nz_v[r, sl] = -zz16
                return carry2

            lax.fori_loop(0, rows * 8, val_body, 0)
            hs = []
            for r in range(rows):
                idx = pix_v.at[r]
                hs.append(pltpu.async_copy(pc_v.at[r], cntmap.at[idx], sem))
                hs.append(pltpu.async_copy(pz_v.at[r], szmap.at[idx], sem))
            for h in hs:
                h.wait()
            plsc.subcore_barrier()
            h1 = pltpu.async_copy(cntmap.at[sl_me],
                                  cnt_hbm.at[b, 0, sl_me], sem)
            h2 = pltpu.async_copy(szmap.at[sl_me],
                                  sz_hbm.at[b, 0, sl_me], sem)
            h1.wait()
            h2.wait()
            plsc.subcore_barrier()
            hs = []
            for r in range(rows):
                idx = pix_v.at[r]
                hs.append(pltpu.async_copy(nc_v.at[r], cntmap.at[idx], sem))
                hs.append(pltpu.async_copy(nz_v.at[r], szmap.at[idx], sem))
            for h in hs:
                h.wait()
            return carry

        lax.fori_loop(0, b_per_core, batch_body, 0)

    return sc_kernel


def kernel(point_clouds, gt_translation_vector, gt_rotation_vector,
           predicted_translation_vector, predicted_rotation_vector,
           gt_rt_matrix, k_matrix, gt_depth_map):
    batch = point_clouds.shape[0]
    n = point_clouds.shape[2]
    pts = jnp.transpose(point_clouds[:, 0], (0, 2, 1))

    smem = pl.BlockSpec(memory_space=pltpu.SMEM)
    pix, z, psum = pl.pallas_call(
        _stage1_body,
        grid=(batch,),
        in_specs=[
            smem,
            smem,
            smem,
            smem,
            pl.BlockSpec((1, 4, n), lambda i: (i, 0, 0)),
        ],
        out_specs=[
            pl.BlockSpec((1, 1, n), lambda i: (i, 0, 0)),
            pl.BlockSpec((1, 1, n), lambda i: (i, 0, 0)),
            pl.BlockSpec((1, 1, 1), lambda i: (i, 0, 0)),
        ],
        out_shape=[
            jax.ShapeDtypeStruct((batch, 1, n), jnp.int32),
            jax.ShapeDtypeStruct((batch, 1, n), jnp.float32),
            jax.ShapeDtypeStruct((batch, 1, 1), jnp.float32),
        ],
    )(gt_rt_matrix, k_matrix, predicted_rotation_vector,
      predicted_translation_vector, pts)

    pix4 = pix.reshape(batch, NSUB, n // NSUB // 128, 128)
    z4 = z.reshape(batch, NSUB, n // NSUB // 128, 128)
    cnt_maps, sz_maps = _make_sc_kernel(batch, n)(pix4, z4)
    cnt3 = cnt_maps.reshape(batch, HPAD, WPAD)
    sz3 = sz_maps.reshape(batch, HPAD, WPAD)

    depth = pl.pallas_call(
        _stage3_body,
        grid=(batch,),
        in_specs=[
            pl.BlockSpec((1, HPAD, WPAD), lambda i: (i, 0, 0)),
            pl.BlockSpec((1, HPAD, WPAD), lambda i: (i, 0, 0)),
            pl.BlockSpec((1, 1, HEIGHT, WIDTH), lambda i: (i, 0, 0, 0)),
        ],
        out_specs=pl.BlockSpec((1, 1, 1), lambda i: (i, 0, 0)),
        out_shape=jax.ShapeDtypeStruct((batch, 1, 1), jnp.float32),
    )(cnt3, sz3, gt_depth_map)

    outs = pl.pallas_call(
        _stage4_body,
        out_shape=[jax.ShapeDtypeStruct((1, 1), jnp.float32)] * 4,
    )(depth, psum / n,
      gt_translation_vector, predicted_translation_vector,
      gt_rotation_vector, predicted_rotation_vector)
    total, tl, dm, pc = (o.reshape(()) for o in outs)
    return (total, tl, dm, pc)
